# baseline (speedup 1.0000x reference)
import functools

import jax
import jax.numpy as jnp
from jax import lax
from jax.experimental import pallas as pl
from jax.experimental.pallas import tpu as pltpu
from jax.experimental.pallas import tpu_sc as plsc

_N_NODES = 100000
_N_PAD = 102144
_SH = 16
_E = 3200000
_EPR = 128 // _SH
_NC = 2
_NS = 16
_NW = _NC * _NS
_E_PER_W = _E // _NW
_CHUNK = 400
_CROWS = _CHUNK // _EPR
_NCHUNK = _E_PER_W // _CHUNK
_NBUF = 2
_ROWS_PER_TILE = _N_PAD // _NS
_ZROWS = 336


def _start_in(b, i, wid, sh_hbm, cut_hbm, recv_hbm, sh_p, cut_v, idx_v, sems):
    base = wid * _E_PER_W + i * _CHUNK
    pltpu.async_copy(sh_hbm.at[pl.ds(base // _EPR, _CROWS)], sh_p.at[b], sems.at[b])
    pltpu.async_copy(cut_hbm.at[pl.ds(base, _CHUNK)], cut_v.at[b], sems.at[b])
    pltpu.async_copy(recv_hbm.at[pl.ds(base, _CHUNK)], idx_v.at[b], sems.at[b])


def _wait_in(b, sh_hbm, cut_hbm, recv_hbm, sh_p, cut_v, idx_v, sems):
    pltpu.make_async_copy(sh_hbm.at[pl.ds(0, _CROWS)], sh_p.at[b], sems.at[b]).wait()
    pltpu.make_async_copy(cut_hbm.at[pl.ds(0, _CHUNK)], cut_v.at[b], sems.at[b]).wait()
    pltpu.make_async_copy(recv_hbm.at[pl.ds(0, _CHUNK)], idx_v.at[b], sems.at[b]).wait()


def _sc_body(sh_hbm, cut_hbm, recv_hbm, out_hbm, sh_p, scaled_v, cut_v, idx_v, acc, sems):
    cid = lax.axis_index("c")
    sid = lax.axis_index("s")
    wid = sid * _NC + cid

    def _zrow(i, carry):
        scaled_v[i, :] = jnp.zeros((_SH,), jnp.float32)
        return carry

    lax.fori_loop(0, _ZROWS, _zrow, None)
    for j in range(_ROWS_PER_TILE // _ZROWS):
        r0 = sid * _ROWS_PER_TILE + j * _ZROWS
        pltpu.sync_copy(scaled_v.at[pl.ds(0, _ZROWS)], acc.at[pl.ds(r0, _ZROWS)])
    plsc.subcore_barrier()

    for b in range(_NBUF):
        _start_in(b, b, wid, sh_hbm, cut_hbm, recv_hbm, sh_p, cut_v, idx_v, sems)

    def _process(i, b):
        _wait_in(b, sh_hbm, cut_hbm, recv_hbm, sh_p, cut_v, idx_v, sems)

        def _mul16(g, c2):
            cvec = cut_v[b, pl.ds(g * _SH, _SH)]
            for j in range(_SH):
                e = g * _SH + j
                r = 2 * g + j // _EPR
                off = (j % _EPR) * _SH
                scaled_v[e, :] = sh_p[b, r, pl.ds(off, _SH)] * cvec[j]
            return c2

        lax.fori_loop(0, _CHUNK // _SH, _mul16, None)
        pltpu.sync_copy(scaled_v, acc.at[idx_v.at[b]], add=True)

        @pl.when(i + _NBUF < _NCHUNK)
        def _refill():
            _start_in(b, i + _NBUF, wid, sh_hbm, cut_hbm, recv_hbm,
                      sh_p, cut_v, idx_v, sems)

    def _pair(k, carry):
        for b in range(_NBUF):
            _process(k * _NBUF + b, b)
        return carry

    lax.fori_loop(0, _NCHUNK // _NBUF, _pair, None)
    for r in range(_NCHUNK - (_NCHUNK // _NBUF) * _NBUF):
        _process(_NCHUNK - 1 + r, (_NCHUNK - 1 + r) % _NBUF)
    plsc.subcore_barrier()

    for j in range(_ROWS_PER_TILE // _ZROWS):
        r0 = sid * _ROWS_PER_TILE + j * _ZROWS
        pltpu.sync_copy(acc.at[pl.ds(r0, _ZROWS)], scaled_v.at[pl.ds(0, _ZROWS)])
        pltpu.sync_copy(
            scaled_v.at[pl.ds(0, _ZROWS)],
            out_hbm.at[pl.ds(cid * _N_PAD + r0, _ZROWS)],
        )


_sc_scatter = functools.partial(
    pl.kernel,
    mesh=plsc.VectorSubcoreMesh(core_axis_name="c", subcore_axis_name="s"),
    out_type=jax.ShapeDtypeStruct((_NC * _N_PAD, _SH), jnp.float32),
    compiler_params=pltpu.CompilerParams(use_tc_tiling_on_sc=False),
    scratch_types=[
        pltpu.VMEM((_NBUF, _CROWS, 128), jnp.float32),
        pltpu.VMEM((_CHUNK, _SH), jnp.float32),
        pltpu.VMEM((_NBUF, _CHUNK), jnp.float32),
        pltpu.VMEM((_NBUF, _CHUNK), jnp.int32),
        pltpu.VMEM_SHARED((_N_PAD, _SH), jnp.float32),
        pltpu.SemaphoreType.DMA((_NBUF,)),
    ],
)(_sc_body)

_RB = 512
_CB = _N_PAD * _SH // _RB
_GB = 64


def _combine_body(inv_ref, p_ref, o_ref):
    o_ref[...] = (p_ref[0] + p_ref[1]) * inv_ref[0]


def kernel(sh_vectors, cutoffs, receivers, inv_avg_num_neighbors):
    sh2 = sh_vectors.reshape(_E // _EPR, 128)
    cut = cutoffs.reshape(_E)
    recv = receivers.astype(jnp.int32)
    part = _sc_scatter(sh2, cut, recv)
    inv_arr = jnp.asarray(inv_avg_num_neighbors, jnp.float32).reshape(1)
    out = pl.pallas_call(
        _combine_body,
        grid=(_RB // _GB,),
        in_specs=[
            pl.BlockSpec(memory_space=pltpu.SMEM),
            pl.BlockSpec((_NC, _GB, _CB), lambda i: (0, i, 0)),
        ],
        out_specs=pl.BlockSpec((_GB, _CB), lambda i: (i, 0)),
        out_shape=jax.ShapeDtypeStruct((_RB, _CB), jnp.float32),
    )(inv_arr, part.reshape(_NC, _RB, _CB))
    return out.reshape(_N_PAD, _SH)[:_N_NODES]


# speedup vs baseline: 1.0258x; 1.0258x over previous
import functools

import jax
import jax.numpy as jnp
from jax import lax
from jax.experimental import pallas as pl
from jax.experimental.pallas import tpu as pltpu
from jax.experimental.pallas import tpu_sc as plsc

_N_NODES = 100000
_N_PAD = 102144
_SH = 16
_E = 3200000
_NC = 2
_NS = 16
_NW = _NC * _NS
_E_PER_W = _E // _NW
_CHUNK = 800
_NCHUNK = _E_PER_W // _CHUNK
_NBUF = 2
_ROWS_PER_TILE = _N_PAD // _NS
_ZROWS = 456


def _start_in(b, i, wid, sh_hbm_r, cut_hbm, recv_hbm, sh_v, cut_v, idx_v, sems):
    base = wid * _E_PER_W + i * _CHUNK
    pltpu.async_copy(sh_hbm_r.at[pl.ds(base, _CHUNK)], sh_v.at[b], sems.at[b])
    pltpu.async_copy(cut_hbm.at[pl.ds(base, _CHUNK)], cut_v.at[b], sems.at[b])
    pltpu.async_copy(recv_hbm.at[pl.ds(base, _CHUNK)], idx_v.at[b], sems.at[b])


def _wait_in(b, sh_hbm_r, cut_hbm, recv_hbm, sh_v, cut_v, idx_v, sems):
    pltpu.make_async_copy(sh_hbm_r.at[pl.ds(0, _CHUNK)], sh_v.at[b], sems.at[b]).wait()
    pltpu.make_async_copy(cut_hbm.at[pl.ds(0, _CHUNK)], cut_v.at[b], sems.at[b]).wait()
    pltpu.make_async_copy(recv_hbm.at[pl.ds(0, _CHUNK)], idx_v.at[b], sems.at[b]).wait()


def _sc_body(sh_hbm, cut_hbm, recv_hbm, out_hbm, sh_v, cut_v, idx_v, acc, sems):
    cid = lax.axis_index("c")
    sid = lax.axis_index("s")
    wid = sid * _NC + cid
    sh_hbm_r = sh_hbm
    out_hbm_r = out_hbm

    def _zrow(i, carry):
        sh_v[0, i, :] = jnp.zeros((_SH,), jnp.float32)
        return carry

    lax.fori_loop(0, _ZROWS, _zrow, None)
    for j in range(_ROWS_PER_TILE // _ZROWS):
        r0 = sid * _ROWS_PER_TILE + j * _ZROWS
        pltpu.sync_copy(sh_v.at[0, pl.ds(0, _ZROWS)], acc.at[pl.ds(r0, _ZROWS)])
    plsc.subcore_barrier()

    for b in range(_NBUF):
        _start_in(b, b, wid, sh_hbm_r, cut_hbm, recv_hbm, sh_v, cut_v, idx_v, sems)

    def _process(i, b):
        _wait_in(b, sh_hbm_r, cut_hbm, recv_hbm, sh_v, cut_v, idx_v, sems)

        def _mul16(g, c2):
            cvec = cut_v[b, pl.ds(g * _SH, _SH)]
            for j in range(_SH):
                e = g * _SH + j
                sh_v[b, e, :] = sh_v[b, e, :] * cvec[j]
            return c2

        lax.fori_loop(0, _CHUNK // _SH, _mul16, None)
        pltpu.sync_copy(sh_v.at[b], acc.at[idx_v.at[b]], add=True)

        @pl.when(i + _NBUF < _NCHUNK)
        def _refill():
            _start_in(b, i + _NBUF, wid, sh_hbm_r, cut_hbm, recv_hbm,
                      sh_v, cut_v, idx_v, sems)

    def _pair(k, carry):
        for b in range(_NBUF):
            _process(k * _NBUF + b, b)
        return carry

    lax.fori_loop(0, _NCHUNK // _NBUF, _pair, None)
    for r in range(_NCHUNK - (_NCHUNK // _NBUF) * _NBUF):
        _process(_NCHUNK - 1 + r, (_NCHUNK - 1 + r) % _NBUF)
    plsc.subcore_barrier()

    for j in range(_ROWS_PER_TILE // _ZROWS):
        r0 = sid * _ROWS_PER_TILE + j * _ZROWS
        pltpu.sync_copy(acc.at[pl.ds(r0, _ZROWS)], sh_v.at[0, pl.ds(0, _ZROWS)])
        pltpu.sync_copy(
            sh_v.at[0, pl.ds(0, _ZROWS)],
            out_hbm_r.at[pl.ds(cid * _N_PAD + r0, _ZROWS)],
        )


_sc_scatter = functools.partial(
    pl.kernel,
    mesh=plsc.VectorSubcoreMesh(core_axis_name="c", subcore_axis_name="s"),
    out_type=jax.ShapeDtypeStruct((_NC * _N_PAD, _SH), jnp.float32),
    compiler_params=pltpu.CompilerParams(use_tc_tiling_on_sc=False),
    scratch_types=[
        pltpu.VMEM((_NBUF, _CHUNK, _SH), jnp.float32),
        pltpu.VMEM((_NBUF, _CHUNK), jnp.float32),
        pltpu.VMEM((_NBUF, _CHUNK), jnp.int32),
        pltpu.VMEM_SHARED((_N_PAD, _SH), jnp.float32),
        pltpu.SemaphoreType.DMA((_NBUF,)),
    ],
)(_sc_body)

_RB = 512
_CB = _N_PAD * _SH // _RB
_GB = 64


def _combine_body(inv_ref, p_ref, o_ref):
    o_ref[...] = (p_ref[0] + p_ref[1]) * inv_ref[0]


def kernel(sh_vectors, cutoffs, receivers, inv_avg_num_neighbors):
    cut = cutoffs.reshape(_E)
    recv = receivers.astype(jnp.int32)
    part = _sc_scatter(sh_vectors, cut, recv)
    inv_arr = jnp.asarray(inv_avg_num_neighbors, jnp.float32).reshape(1)
    out = pl.pallas_call(
        _combine_body,
        grid=(_RB // _GB,),
        in_specs=[
            pl.BlockSpec(memory_space=pltpu.SMEM),
            pl.BlockSpec((_NC, _GB, _CB), lambda i: (0, i, 0)),
        ],
        out_specs=pl.BlockSpec((_GB, _CB), lambda i: (i, 0)),
        out_shape=jax.ShapeDtypeStruct((_RB, _CB), jnp.float32),
    )(inv_arr, part.reshape(_NC, _RB, _CB))
    return out.reshape(_N_PAD, _SH)[:_N_NODES]
